# baseline (speedup 1.0000x reference)
import functools
import math

import jax
import jax.numpy as jnp
from jax import lax
from jax.experimental import pallas as pl
from jax.experimental.pallas import tpu as pltpu
from jax.experimental.pallas import tpu_sc as plsc

_VOCAB = 1000000
_EMB = 64
_B = 4096
_L = 200
_N = _B * _L

_NC = 2
_NS = 16
_NW = _NC * _NS
_PER_W = _N // _NW
_C = 128
_NCH = _PER_W // _C
_SCALE = math.sqrt(float(_EMB))


def _scale_buf(buf):
    def row_body(r, _):
        for j in range(_EMB // 16):
            sl = pl.ds(j * 16, 16)
            buf[r, sl] = buf[r, sl] * _SCALE
        return 0
    lax.fori_loop(0, _C, row_body, 0)


def _make_sc_kernel():
    mesh = plsc.VectorSubcoreMesh(core_axis_name="c", subcore_axis_name="s")

    @functools.partial(
        pl.kernel,
        mesh=mesh,
        out_type=jax.ShapeDtypeStruct((_N, _EMB), jnp.float32),
        scratch_types=[
            pltpu.VMEM((_NCH, _C), jnp.int32),
            pltpu.VMEM((_C, _EMB), jnp.float32),
            pltpu.VMEM((_C, _EMB), jnp.float32),
            pltpu.SemaphoreType.DMA,
            pltpu.SemaphoreType.DMA,
        ],
        compiler_params=pltpu.CompilerParams(use_tc_tiling_on_sc=False),
    )
    def sc_embed(idx_hbm, table_hbm, out_hbm, idx_v, rows0, rows1, g0, g1):
        wid = lax.axis_index("s") * _NC + lax.axis_index("c")
        base = wid * _PER_W

        pltpu.sync_copy(idx_hbm.at[pl.ds(wid * _NCH, _NCH)], idx_v)

        bufs = (rows0, rows1)
        sems = (g0, g1)

        def start_gather(g, b):
            pltpu.async_copy(table_hbm.at[idx_v.at[g]], bufs[b], sems[b])

        def wait_gather(g, b):
            pltpu.make_async_copy(
                table_hbm.at[idx_v.at[g]], bufs[b], sems[b]
            ).wait()

        def process(g, b):
            wait_gather(g, b)
            _scale_buf(bufs[b])
            pltpu.sync_copy(bufs[b], out_hbm.at[pl.ds(base + g * _C, _C)])

        start_gather(0, 0)
        start_gather(1, 1)

        def body(i, _):
            for b in range(2):
                g = 2 * i + b
                process(g, b)
                start_gather(g + 2, b)
            return 0

        lax.fori_loop(0, _NCH // 2 - 1, body, 0)

        process(_NCH - 2, 0)
        process(_NCH - 1, 1)

    return sc_embed


_sc_embed = _make_sc_kernel()


def kernel(tokens, table):
    idx = tokens.reshape(_NW * _NCH, _C)
    out = _sc_embed(idx, table)
    return out.reshape(_B, _L, _EMB)
